# baseline (speedup 1.0000x reference)
import jax
import jax.numpy as jnp
from jax import lax
from jax.experimental import pallas as pl
from jax.experimental.pallas import tpu as pltpu
from jax.experimental.pallas import tpu_sc as plsc

B = 4096
NC, NS = 2, 16
NW = NC * NS
BPW = B // NW
MF_DIM = 64
EMB_HALF = 128
ITEM_EMB_DIM = 768
BLK = 512
CHUNK = 16


def _sc_body(user_hbm, item_hbm, ue_gmf_hbm, ie_gmf_hbm,
             ue_mlp_hbm, ie_mlp_hbm,
             gu_out, gi_out, mu_out, mi_out,
             idx_u, idx_i,
             bu, bi, rows_u, rows_i, buf_mu, buf_mi, sem, semg):
    wid = lax.axis_index("s") * NC + lax.axis_index("c")
    base = wid * BPW
    pltpu.sync_copy(user_hbm.at[pl.ds(base, BPW)], idx_u)
    pltpu.sync_copy(item_hbm.at[pl.ds(base, BPW)], idx_i)
    cm1 = pltpu.async_copy(ue_mlp_hbm.at[idx_u], buf_mu, sem)
    cm2 = pltpu.async_copy(ie_mlp_hbm.at[idx_i], buf_mi, sem)
    for c0 in range(0, BPW, CHUNK):
        uv = idx_u[pl.ds(c0, CHUNK)]
        iv = idx_i[pl.ds(c0, CHUNK)]
        cs = []
        for k in range(CHUNK):
            cs.append(pltpu.async_copy(
                ue_gmf_hbm.at[pl.ds(uv[k], 1), :],
                rows_u.at[pl.ds(c0 + k, 1), :], semg))
            cs.append(pltpu.async_copy(
                ie_gmf_hbm.at[pl.ds(iv[k], 1), :],
                rows_i.at[pl.ds(c0 + k, 1), :], semg))
        for c in cs:
            c.wait()
    cm1.wait()
    cm2.wait()
    pltpu.sync_copy(rows_u, gu_out.at[pl.ds(base, BPW)])
    pltpu.sync_copy(rows_i, gi_out.at[pl.ds(base, BPW)])
    pltpu.sync_copy(buf_mu, mu_out.at[pl.ds(base, BPW)])
    pltpu.sync_copy(buf_mi, mi_out.at[pl.ds(base, BPW)])


def _sc_gather(user, item, ue_gmf, ie_gmf, ue_mlp, ie_mlp):
    fn = pl.kernel(
        _sc_body,
        mesh=plsc.VectorSubcoreMesh(core_axis_name="c", subcore_axis_name="s"),
        compiler_params=pltpu.CompilerParams(use_tc_tiling_on_sc=True),
        out_type=[
            jax.ShapeDtypeStruct((B, MF_DIM), jnp.float32),
            jax.ShapeDtypeStruct((B, MF_DIM), jnp.float32),
            jax.ShapeDtypeStruct((B, EMB_HALF), jnp.float32),
            jax.ShapeDtypeStruct((B, EMB_HALF), jnp.float32),
        ],
        scratch_types=[
            pltpu.VMEM((BPW,), jnp.int32),
            pltpu.VMEM((BPW,), jnp.int32),
            pltpu.VMEM((CHUNK, 8, MF_DIM), jnp.float32),
            pltpu.VMEM((CHUNK, 8, MF_DIM), jnp.float32),
            pltpu.VMEM((BPW, MF_DIM), jnp.float32),
            pltpu.VMEM((BPW, MF_DIM), jnp.float32),
            pltpu.VMEM((BPW, EMB_HALF), jnp.float32),
            pltpu.VMEM((BPW, EMB_HALF), jnp.float32),
            pltpu.SemaphoreType.DMA,
            pltpu.SemaphoreType.DMA,
        ],
    )
    return fn(user, item, ue_gmf, ie_gmf, ue_mlp, ie_mlp)


TBLK = 2048
NROWS = 100000


def _tr_body(in_ref, out_ref):
    out_ref[...] = in_ref[...].T


def _tc_transpose(xt):
    grid = ((NROWS + TBLK - 1) // TBLK,)
    return pl.pallas_call(
        _tr_body,
        grid=grid,
        in_specs=[pl.BlockSpec((MF_DIM, TBLK), lambda j: (0, j))],
        out_specs=pl.BlockSpec((TBLK, MF_DIM), lambda j: (j, 0)),
        out_shape=jax.ShapeDtypeStruct((NROWS, MF_DIM), jnp.float32),
    )(xt)


def _tc_body(gu_ref, gi_ref, mu_ref, mi_ref, emb_ref,
             w1a_ref, w1b_ref, w1c_ref, b1_ref,
             w2_ref, b2_ref, w3_ref, b3_ref,
             wpg_ref, wph_ref, bp_ref, out_ref):
    f32 = jnp.float32
    bf = jnp.bfloat16
    xu = mu_ref[...].astype(bf)
    xi = mi_ref[...].astype(bf)
    xe = emb_ref[...].astype(bf)
    h = (jnp.dot(xu, w1a_ref[...], preferred_element_type=f32)
         + jnp.dot(xi, w1b_ref[...], preferred_element_type=f32)
         + jnp.dot(xe, w1c_ref[...], preferred_element_type=f32))
    h = jnp.maximum(h + b1_ref[...][None, :], 0.0)
    h = jnp.dot(h.astype(bf), w2_ref[...], preferred_element_type=f32)
    h = jnp.maximum(h + b2_ref[...][None, :], 0.0)
    h = jnp.dot(h.astype(bf), w3_ref[...], preferred_element_type=f32)
    h = jnp.maximum(h + b3_ref[...][None, :], 0.0)
    g = (gu_ref[...] * gi_ref[...]).astype(bf)
    pred = (jnp.dot(g, wpg_ref[...].astype(bf), preferred_element_type=f32)
            + jnp.dot(h.astype(bf), wph_ref[...].astype(bf),
                      preferred_element_type=f32))
    out_ref[...] = pred + bp_ref[...]


def _tc_dense(gu, gi, mu, mi, emb, w1a, w1b, w1c, b1, w2, b2, w3, b3,
              wpg, wph, bp):
    grid = (B // BLK,)
    full = lambda shape: pl.BlockSpec(shape, lambda i: tuple(0 for _ in shape))
    return pl.pallas_call(
        _tc_body,
        grid=grid,
        in_specs=[
            pl.BlockSpec((BLK, MF_DIM), lambda i: (i, 0)),
            pl.BlockSpec((BLK, MF_DIM), lambda i: (i, 0)),
            pl.BlockSpec((BLK, EMB_HALF), lambda i: (i, 0)),
            pl.BlockSpec((BLK, EMB_HALF), lambda i: (i, 0)),
            pl.BlockSpec((BLK, ITEM_EMB_DIM), lambda i: (i, 0)),
            full((EMB_HALF, 1024)),
            full((EMB_HALF, 1024)),
            full((ITEM_EMB_DIM, 1024)),
            full((1024,)),
            full((1024, 512)),
            full((512,)),
            full((512, 256)),
            full((256,)),
            full((MF_DIM, 1)),
            full((256, 1)),
            full((1,)),
        ],
        out_specs=pl.BlockSpec((BLK, 1), lambda i: (i, 0)),
        out_shape=jax.ShapeDtypeStruct((B, 1), jnp.float32),
    )(gu, gi, mu, mi, emb, w1a, w1b, w1c, b1, w2, b2, w3, b3, wpg, wph, bp)


def kernel(user, item, item_embedding, ue_gmf, ie_gmf, ue_mlp, ie_mlp,
           W1, b1, W2, b2, W3, b3, Wp, bp):
    pe_u = _tc_transpose(ue_gmf.T)
    pe_i = _tc_transpose(ie_gmf.T)
    gu, gi, mu, mi = _sc_gather(user, item, pe_u, pe_i, ue_mlp, ie_mlp)
    bf = jnp.bfloat16
    out = _tc_dense(
        gu, gi, mu, mi, item_embedding,
        W1[:EMB_HALF].astype(bf), W1[EMB_HALF:2 * EMB_HALF].astype(bf),
        W1[2 * EMB_HALF:].astype(bf), b1,
        W2.astype(bf), b2, W3.astype(bf), b3,
        Wp[:MF_DIM], Wp[MF_DIM:], bp)
    return out[:, 0]


# speedup vs baseline: 1.1856x; 1.1856x over previous
import jax
import jax.numpy as jnp
from jax import lax
from jax.experimental import pallas as pl
from jax.experimental.pallas import tpu as pltpu
from jax.experimental.pallas import tpu_sc as plsc

B = 4096
NC, NS = 2, 16
NW = NC * NS
BPW = B // NW
MF_DIM = 64
EMB_HALF = 128
ITEM_EMB_DIM = 768
BLK = 512
CHUNK = 16


def _sc_mlp_body(user_hbm, item_hbm, ue_mlp_hbm, ie_mlp_hbm,
                 mu_out, mi_out, idx_u, idx_i, buf_mu, buf_mi, sem):
    wid = lax.axis_index("s") * NC + lax.axis_index("c")
    base = wid * BPW
    pltpu.sync_copy(user_hbm.at[pl.ds(base, BPW)], idx_u)
    pltpu.sync_copy(item_hbm.at[pl.ds(base, BPW)], idx_i)
    cm1 = pltpu.async_copy(ue_mlp_hbm.at[idx_u], buf_mu, sem)
    cm2 = pltpu.async_copy(ie_mlp_hbm.at[idx_i], buf_mi, sem)
    cm1.wait()
    cm2.wait()
    pltpu.sync_copy(buf_mu, mu_out.at[pl.ds(base, BPW)])
    pltpu.sync_copy(buf_mi, mi_out.at[pl.ds(base, BPW)])


def _sc_mlp_gather(user, item, ue_mlp, ie_mlp):
    fn = pl.kernel(
        _sc_mlp_body,
        mesh=plsc.VectorSubcoreMesh(core_axis_name="c", subcore_axis_name="s"),
        compiler_params=pltpu.CompilerParams(use_tc_tiling_on_sc=True),
        out_type=[
            jax.ShapeDtypeStruct((B, EMB_HALF), jnp.float32),
            jax.ShapeDtypeStruct((B, EMB_HALF), jnp.float32),
        ],
        scratch_types=[
            pltpu.VMEM((BPW,), jnp.int32),
            pltpu.VMEM((BPW,), jnp.int32),
            pltpu.VMEM((BPW, EMB_HALF), jnp.float32),
            pltpu.VMEM((BPW, EMB_HALF), jnp.float32),
            pltpu.SemaphoreType.DMA,
        ],
    )
    return fn(user, item, ue_mlp, ie_mlp)


def _sc_gmf_body(user_hbm, item_hbm, ue_gmf_hbm, ie_gmf_hbm,
                 gu_out, gi_out, idx_u, idx_i, rows_u, rows_i, semg):
    wid = lax.axis_index("s") * NC + lax.axis_index("c")
    base = wid * BPW
    pltpu.sync_copy(user_hbm.at[pl.ds(base, BPW)], idx_u)
    pltpu.sync_copy(item_hbm.at[pl.ds(base, BPW)], idx_i)
    for c0 in range(0, BPW, CHUNK):
        uv = idx_u[pl.ds(c0, CHUNK)]
        iv = idx_i[pl.ds(c0, CHUNK)]
        cs = []
        for k in range(CHUNK):
            cs.append(pltpu.async_copy(
                ue_gmf_hbm.at[pl.ds(uv[k], 1), :],
                rows_u.at[pl.ds(c0 + k, 1), :], semg))
            cs.append(pltpu.async_copy(
                ie_gmf_hbm.at[pl.ds(iv[k], 1), :],
                rows_i.at[pl.ds(c0 + k, 1), :], semg))
        for c in cs:
            c.wait()
    pltpu.sync_copy(rows_u, gu_out.at[pl.ds(base, BPW)])
    pltpu.sync_copy(rows_i, gi_out.at[pl.ds(base, BPW)])


def _sc_gmf_gather(user, item, ue_gmf, ie_gmf):
    fn = pl.kernel(
        _sc_gmf_body,
        mesh=plsc.VectorSubcoreMesh(core_axis_name="c", subcore_axis_name="s"),
        compiler_params=pltpu.CompilerParams(use_tc_tiling_on_sc=True),
        out_type=[
            jax.ShapeDtypeStruct((B, MF_DIM), jnp.float32),
            jax.ShapeDtypeStruct((B, MF_DIM), jnp.float32),
        ],
        scratch_types=[
            pltpu.VMEM((BPW,), jnp.int32),
            pltpu.VMEM((BPW,), jnp.int32),
            pltpu.VMEM((BPW, MF_DIM), jnp.float32),
            pltpu.VMEM((BPW, MF_DIM), jnp.float32),
            pltpu.SemaphoreType.DMA,
        ],
    )
    return fn(user, item, ue_gmf, ie_gmf)


def _tc_mlp_body(mu_ref, mi_ref, emb_ref,
                 w1a_ref, w1b_ref, w1c_ref, b1_ref,
                 w2_ref, b2_ref, w3_ref, b3_ref,
                 wph_ref, out_ref):
    f32 = jnp.float32
    bf = jnp.bfloat16
    xu = mu_ref[...].astype(bf)
    xi = mi_ref[...].astype(bf)
    xe = emb_ref[...].astype(bf)
    h = (jnp.dot(xu, w1a_ref[...], preferred_element_type=f32)
         + jnp.dot(xi, w1b_ref[...], preferred_element_type=f32)
         + jnp.dot(xe, w1c_ref[...], preferred_element_type=f32))
    h = jnp.maximum(h + b1_ref[...][None, :], 0.0)
    h = jnp.dot(h.astype(bf), w2_ref[...], preferred_element_type=f32)
    h = jnp.maximum(h + b2_ref[...][None, :], 0.0)
    h = jnp.dot(h.astype(bf), w3_ref[...], preferred_element_type=f32)
    h = jnp.maximum(h + b3_ref[...][None, :], 0.0)
    out_ref[...] = jnp.dot(h.astype(bf), wph_ref[...].astype(bf),
                           preferred_element_type=f32)


def _tc_mlp(mu, mi, emb, w1a, w1b, w1c, b1, w2, b2, w3, b3, wph):
    grid = (B // BLK,)
    full = lambda shape: pl.BlockSpec(shape, lambda i: tuple(0 for _ in shape))
    return pl.pallas_call(
        _tc_mlp_body,
        grid=grid,
        in_specs=[
            pl.BlockSpec((BLK, EMB_HALF), lambda i: (i, 0)),
            pl.BlockSpec((BLK, EMB_HALF), lambda i: (i, 0)),
            pl.BlockSpec((BLK, ITEM_EMB_DIM), lambda i: (i, 0)),
            full((EMB_HALF, 1024)),
            full((EMB_HALF, 1024)),
            full((ITEM_EMB_DIM, 1024)),
            full((1024,)),
            full((1024, 512)),
            full((512,)),
            full((512, 256)),
            full((256,)),
            full((256, 1)),
        ],
        out_specs=pl.BlockSpec((BLK, 1), lambda i: (i, 0)),
        out_shape=jax.ShapeDtypeStruct((B, 1), jnp.float32),
    )(mu, mi, emb, w1a, w1b, w1c, b1, w2, b2, w3, b3, wph)


def _tc_gmf_body(gu_ref, gi_ref, pm_ref, wpg_ref, bp_ref, out_ref):
    f32 = jnp.float32
    bf = jnp.bfloat16
    g = (gu_ref[...] * gi_ref[...]).astype(bf)
    pred = jnp.dot(g, wpg_ref[...].astype(bf), preferred_element_type=f32)
    out_ref[...] = pred + pm_ref[...] + bp_ref[...]


def _tc_gmf(gu, gi, pm, wpg, bp):
    grid = (B // 2048,)
    full = lambda shape: pl.BlockSpec(shape, lambda i: tuple(0 for _ in shape))
    return pl.pallas_call(
        _tc_gmf_body,
        grid=grid,
        in_specs=[
            pl.BlockSpec((2048, MF_DIM), lambda i: (i, 0)),
            pl.BlockSpec((2048, MF_DIM), lambda i: (i, 0)),
            pl.BlockSpec((2048, 1), lambda i: (i, 0)),
            full((MF_DIM, 1)),
            full((1,)),
        ],
        out_specs=pl.BlockSpec((2048, 1), lambda i: (i, 0)),
        out_shape=jax.ShapeDtypeStruct((B, 1), jnp.float32),
    )(gu, gi, pm, wpg, bp)


def kernel(user, item, item_embedding, ue_gmf, ie_gmf, ue_mlp, ie_mlp,
           W1, b1, W2, b2, W3, b3, Wp, bp):
    bf = jnp.bfloat16
    mu, mi = _sc_mlp_gather(user, item, ue_mlp, ie_mlp)
    pm = _tc_mlp(
        mu, mi, item_embedding,
        W1[:EMB_HALF].astype(bf), W1[EMB_HALF:2 * EMB_HALF].astype(bf),
        W1[2 * EMB_HALF:].astype(bf), b1,
        W2.astype(bf), b2, W3.astype(bf), b3, Wp[MF_DIM:])
    gu, gi = _sc_gmf_gather(user, item, ue_gmf, ie_gmf)
    out = _tc_gmf(gu, gi, pm, Wp[:MF_DIM], bp)
    return out[:, 0]


# speedup vs baseline: 1.1882x; 1.0022x over previous
import jax
import jax.numpy as jnp
from jax import lax
from jax.experimental import pallas as pl
from jax.experimental.pallas import tpu as pltpu
from jax.experimental.pallas import tpu_sc as plsc

B = 4096
NC, NS = 2, 16
NW = NC * NS
BPW = B // NW
MF_DIM = 64
EMB_HALF = 128
ITEM_EMB_DIM = 768
BLK = 512
CHUNK = 32


def _sc_mlp_body(user_hbm, item_hbm, ue_mlp_hbm, ie_mlp_hbm,
                 mu_out, mi_out, idx_u, idx_i, buf_mu, buf_mi, sem):
    wid = lax.axis_index("s") * NC + lax.axis_index("c")
    base = wid * BPW
    pltpu.sync_copy(user_hbm.at[pl.ds(base, BPW)], idx_u)
    pltpu.sync_copy(item_hbm.at[pl.ds(base, BPW)], idx_i)
    cm1 = pltpu.async_copy(ue_mlp_hbm.at[idx_u], buf_mu, sem)
    cm2 = pltpu.async_copy(ie_mlp_hbm.at[idx_i], buf_mi, sem)
    cm1.wait()
    cm2.wait()
    pltpu.sync_copy(buf_mu, mu_out.at[pl.ds(base, BPW)])
    pltpu.sync_copy(buf_mi, mi_out.at[pl.ds(base, BPW)])


def _sc_mlp_gather(user, item, ue_mlp, ie_mlp):
    fn = pl.kernel(
        _sc_mlp_body,
        mesh=plsc.VectorSubcoreMesh(core_axis_name="c", subcore_axis_name="s"),
        compiler_params=pltpu.CompilerParams(use_tc_tiling_on_sc=True),
        out_type=[
            jax.ShapeDtypeStruct((B, EMB_HALF), jnp.float32),
            jax.ShapeDtypeStruct((B, EMB_HALF), jnp.float32),
        ],
        scratch_types=[
            pltpu.VMEM((BPW,), jnp.int32),
            pltpu.VMEM((BPW,), jnp.int32),
            pltpu.VMEM((BPW, EMB_HALF), jnp.float32),
            pltpu.VMEM((BPW, EMB_HALF), jnp.float32),
            pltpu.SemaphoreType.DMA,
        ],
    )
    return fn(user, item, ue_mlp, ie_mlp)


def _sc_gmf_body(user_hbm, item_hbm, ue_gmf_hbm, ie_gmf_hbm,
                 gu_out, gi_out, idx_u, idx_i, rows_u, rows_i, semg):
    wid = lax.axis_index("s") * NC + lax.axis_index("c")
    base = wid * BPW
    pltpu.sync_copy(user_hbm.at[pl.ds(base, BPW)], idx_u)
    pltpu.sync_copy(item_hbm.at[pl.ds(base, BPW)], idx_i)
    for c0 in range(0, BPW, CHUNK):
        uv = idx_u[pl.ds(c0, CHUNK)]
        iv = idx_i[pl.ds(c0, CHUNK)]
        cs = []
        for k in range(CHUNK):
            cs.append(pltpu.async_copy(
                ue_gmf_hbm.at[pl.ds(uv[k], 1), :],
                rows_u.at[pl.ds(c0 + k, 1), :], semg))
            cs.append(pltpu.async_copy(
                ie_gmf_hbm.at[pl.ds(iv[k], 1), :],
                rows_i.at[pl.ds(c0 + k, 1), :], semg))
        for c in cs:
            c.wait()
    pltpu.sync_copy(rows_u, gu_out.at[pl.ds(base, BPW)])
    pltpu.sync_copy(rows_i, gi_out.at[pl.ds(base, BPW)])


def _sc_gmf_gather(user, item, ue_gmf, ie_gmf):
    fn = pl.kernel(
        _sc_gmf_body,
        mesh=plsc.VectorSubcoreMesh(core_axis_name="c", subcore_axis_name="s"),
        compiler_params=pltpu.CompilerParams(use_tc_tiling_on_sc=True),
        out_type=[
            jax.ShapeDtypeStruct((B, MF_DIM), jnp.float32),
            jax.ShapeDtypeStruct((B, MF_DIM), jnp.float32),
        ],
        scratch_types=[
            pltpu.VMEM((BPW,), jnp.int32),
            pltpu.VMEM((BPW,), jnp.int32),
            pltpu.VMEM((BPW, MF_DIM), jnp.float32),
            pltpu.VMEM((BPW, MF_DIM), jnp.float32),
            pltpu.SemaphoreType.DMA,
        ],
    )
    return fn(user, item, ue_gmf, ie_gmf)


def _tc_mlp_body(mu_ref, mi_ref, emb_ref,
                 w1a_ref, w1b_ref, w1c_ref, b1_ref,
                 w2_ref, b2_ref, w3_ref, b3_ref,
                 wph_ref, out_ref):
    f32 = jnp.float32
    bf = jnp.bfloat16
    xu = mu_ref[...].astype(bf)
    xi = mi_ref[...].astype(bf)
    xe = emb_ref[...].astype(bf)
    h = (jnp.dot(xu, w1a_ref[...], preferred_element_type=f32)
         + jnp.dot(xi, w1b_ref[...], preferred_element_type=f32)
         + jnp.dot(xe, w1c_ref[...], preferred_element_type=f32))
    h = jnp.maximum(h + b1_ref[...][None, :], 0.0)
    h = jnp.dot(h.astype(bf), w2_ref[...], preferred_element_type=f32)
    h = jnp.maximum(h + b2_ref[...][None, :], 0.0)
    h = jnp.dot(h.astype(bf), w3_ref[...], preferred_element_type=f32)
    h = jnp.maximum(h + b3_ref[...][None, :], 0.0)
    out_ref[...] = jnp.dot(h.astype(bf), wph_ref[...].astype(bf),
                           preferred_element_type=f32)


def _tc_mlp(mu, mi, emb, w1a, w1b, w1c, b1, w2, b2, w3, b3, wph):
    grid = (B // BLK,)
    full = lambda shape: pl.BlockSpec(shape, lambda i: tuple(0 for _ in shape))
    return pl.pallas_call(
        _tc_mlp_body,
        grid=grid,
        in_specs=[
            pl.BlockSpec((BLK, EMB_HALF), lambda i: (i, 0)),
            pl.BlockSpec((BLK, EMB_HALF), lambda i: (i, 0)),
            pl.BlockSpec((BLK, ITEM_EMB_DIM), lambda i: (i, 0)),
            full((EMB_HALF, 1024)),
            full((EMB_HALF, 1024)),
            full((ITEM_EMB_DIM, 1024)),
            full((1024,)),
            full((1024, 512)),
            full((512,)),
            full((512, 256)),
            full((256,)),
            full((256, 1)),
        ],
        out_specs=pl.BlockSpec((BLK, 1), lambda i: (i, 0)),
        out_shape=jax.ShapeDtypeStruct((B, 1), jnp.float32),
    )(mu, mi, emb, w1a, w1b, w1c, b1, w2, b2, w3, b3, wph)


def _tc_gmf_body(gu_ref, gi_ref, pm_ref, wpg_ref, bp_ref, out_ref):
    f32 = jnp.float32
    bf = jnp.bfloat16
    g = (gu_ref[...] * gi_ref[...]).astype(bf)
    pred = jnp.dot(g, wpg_ref[...].astype(bf), preferred_element_type=f32)
    out_ref[...] = pred + pm_ref[...] + bp_ref[...]


def _tc_gmf(gu, gi, pm, wpg, bp):
    grid = (B // 2048,)
    full = lambda shape: pl.BlockSpec(shape, lambda i: tuple(0 for _ in shape))
    return pl.pallas_call(
        _tc_gmf_body,
        grid=grid,
        in_specs=[
            pl.BlockSpec((2048, MF_DIM), lambda i: (i, 0)),
            pl.BlockSpec((2048, MF_DIM), lambda i: (i, 0)),
            pl.BlockSpec((2048, 1), lambda i: (i, 0)),
            full((MF_DIM, 1)),
            full((1,)),
        ],
        out_specs=pl.BlockSpec((2048, 1), lambda i: (i, 0)),
        out_shape=jax.ShapeDtypeStruct((B, 1), jnp.float32),
    )(gu, gi, pm, wpg, bp)


def kernel(user, item, item_embedding, ue_gmf, ie_gmf, ue_mlp, ie_mlp,
           W1, b1, W2, b2, W3, b3, Wp, bp):
    bf = jnp.bfloat16
    mu, mi = _sc_mlp_gather(user, item, ue_mlp, ie_mlp)
    pm = _tc_mlp(
        mu, mi, item_embedding,
        W1[:EMB_HALF].astype(bf), W1[EMB_HALF:2 * EMB_HALF].astype(bf),
        W1[2 * EMB_HALF:].astype(bf), b1,
        W2.astype(bf), b2, W3.astype(bf), b3, Wp[MF_DIM:])
    gu, gi = _sc_gmf_gather(user, item, ue_gmf, ie_gmf)
    out = _tc_gmf(gu, gi, pm, Wp[:MF_DIM], bp)
    return out[:, 0]
